# baseline (speedup 1.0000x reference)
import functools

import jax
import jax.numpy as jnp
from jax import lax
from jax.experimental import pallas as pl
from jax.experimental.pallas import tpu as pltpu
from jax.experimental.pallas import tpu_sc as plsc

_N = 10000
_E = 320000
_F_IN = 128
_H1 = 8
_C1 = 8
_NCLS = 32

_NB = 1000
_NC = 2
_NS = 16
_NW = _NC * _NS
_K = 80
_LANES = 16
_NBUF = 4


def _prep1_body(x_ref, w1_ref, s1_ref, d1_ref, e8_ref, g_ref, a_ref, i_ref):
    h = jnp.dot(x_ref[...], w1_ref[...], preferred_element_type=jnp.float32)
    a_s = jnp.dot(h, s1_ref[...], preferred_element_type=jnp.float32)
    a_d = jnp.dot(h, d1_ref[...], preferred_element_type=jnp.float32)
    z8 = jnp.zeros((_NB, 8), jnp.float32)
    z48 = jnp.zeros((_NB, 48), jnp.float32)
    g_ref[...] = jnp.concatenate([a_s, z8, h], axis=1)
    a_ref[...] = jnp.concatenate([a_d, z8], axis=1)
    sm = a_s + a_d
    w = jnp.exp(jnp.where(sm > 0, sm, 0.2 * sm))
    wh = jnp.dot(w, e8_ref[...], preferred_element_type=jnp.float32) * h
    i_ref[...] = jnp.concatenate([0.5 * w, z8, 0.5 * wh, z48], axis=1)


def _prep1(x, W1, S1, D1, E8):
    grid = _N // _NB
    return pl.pallas_call(
        _prep1_body,
        grid=(grid,),
        in_specs=[
            pl.BlockSpec((_NB, _F_IN), lambda i: (i, 0)),
            pl.BlockSpec((_F_IN, _H1 * _C1), lambda i: (0, 0)),
            pl.BlockSpec((_H1 * _C1, _H1), lambda i: (0, 0)),
            pl.BlockSpec((_H1 * _C1, _H1), lambda i: (0, 0)),
            pl.BlockSpec((_H1, _H1 * _C1), lambda i: (0, 0)),
        ],
        out_specs=[
            pl.BlockSpec((_NB, 80), lambda i: (i, 0)),
            pl.BlockSpec((_NB, 16), lambda i: (i, 0)),
            pl.BlockSpec((_NB, 128), lambda i: (i, 0)),
        ],
        out_shape=[
            jax.ShapeDtypeStruct((_N, 80), jnp.float32),
            jax.ShapeDtypeStruct((_N, 16), jnp.float32),
            jax.ShapeDtypeStruct((_N, 128), jnp.float32),
        ],
    )(x, W1, S1, D1, E8)


def _edge_pass(src3, dst3, G, A, I, R, nh, bcast):
    n = G.shape[0]
    nch = src3.shape[1]
    r_off = 624
    r_sz = n - r_off * (_NS - 1)
    assert r_off % 8 == 0 and r_sz % 8 == 0 and r_sz >= r_off
    mesh = plsc.VectorSubcoreMesh(core_axis_name="c", subcore_axis_name="s")

    @functools.partial(
        pl.kernel,
        out_type=jax.ShapeDtypeStruct((_NC, n, 128), jnp.float32),
        mesh=mesh,
        compiler_params=pltpu.CompilerParams(
            needs_layout_passes=False, use_tc_tiling_on_sc=False),
        scratch_types=[
            pltpu.VMEM((nch, _K), jnp.int32),
            pltpu.VMEM((nch, _K), jnp.int32),
            pltpu.VMEM((_NBUF, _K, R), jnp.float32),
            pltpu.VMEM((_NBUF, _K, 16), jnp.float32),
            pltpu.VMEM((_NBUF, _K, R), jnp.float32),
            pltpu.VMEM((_K * _LANES,), jnp.float32),
            pltpu.VMEM_SHARED((n, R), jnp.float32),
            pltpu.SemaphoreType.DMA((_NBUF,)),
            pltpu.SemaphoreType.DMA((_NBUF,)),
            pltpu.SemaphoreType.DMA((_NBUF,)),
        ],
    )
    def ek(src_hbm, dst_hbm, g_hbm, a_hbm, i_hbm, out_hbm,
           sidx, didx, grows, arows, mrows, wbuf, acc,
           semg, semr, semd):
        c = lax.axis_index("c")
        s = lax.axis_index("s")
        wid = s * _NC + c
        pltpu.sync_copy(i_hbm.at[pl.ds(s * r_off, r_sz), pl.ds(0, R)],
                        acc.at[pl.ds(s * r_off, r_sz)])
        pltpu.sync_copy(src_hbm.at[wid], sidx)
        pltpu.sync_copy(dst_hbm.at[wid], didx)
        plsc.subcore_barrier()
        iota = lax.iota(jnp.int32, _LANES)

        def issue_gather(i, b):
            pltpu.async_copy(g_hbm.at[sidx.at[i]], grows.at[b], semg.at[b])
            pltpu.async_copy(a_hbm.at[didx.at[i]], arows.at[b], semr.at[b])

        for p in range(_NBUF - 1):
            issue_gather(p, p)

        def chunk(i, carry):
            b = lax.rem(i, _NBUF)

            @pl.when(i + _NBUF - 1 < nch)
            def _():
                issue_gather(i + _NBUF - 1, lax.rem(i + _NBUF - 1, _NBUF))

            pltpu.make_async_copy(
                g_hbm.at[sidx.at[i]], grows.at[b], semg.at[b]).wait()
            pltpu.make_async_copy(
                a_hbm.at[didx.at[i]], arows.at[b], semr.at[b]).wait()

            @pl.when(i >= _NBUF)
            def _():
                pltpu.make_async_copy(
                    mrows.at[b], acc.at[didx.at[i]], semd.at[b]).wait()

            @plsc.parallel_loop(0, _K, unroll=8)
            def edge(e):
                va = grows[b, e, pl.ds(0, 16)]
                vd = arows[b, e, pl.ds(0, 16)]
                sm = va + vd
                w = jnp.exp(jnp.maximum(sm, 0.2 * sm))
                mrows[b, e, pl.ds(0, 16)] = w
                wbuf[pl.ds(e * _LANES, _LANES)] = w
                ebase = jnp.full((_LANES,), e * _LANES, jnp.int32)
                for j in range(nh):
                    if bcast:
                        col = ebase
                    else:
                        col = ebase + (iota >> 3) + (2 * j)
                    wj = plsc.load_gather(wbuf, [col])
                    hj = grows[b, e, pl.ds(16 + 16 * j, 16)]
                    mrows[b, e, pl.ds(16 + 16 * j, 16)] = hj * wj

            pltpu.async_copy(mrows.at[b], acc.at[didx.at[i]],
                             semd.at[b], add=True)
            return carry

        lax.fori_loop(0, nch, chunk, 0)
        for p in range(_NBUF):
            pltpu.make_async_copy(mrows.at[p], acc.at[didx.at[p]],
                                  semd.at[p]).wait()
        plsc.subcore_barrier()
        pltpu.sync_copy(acc.at[pl.ds(s * r_off, r_sz)],
                        out_hbm.at[c, pl.ds(s * r_off, r_sz), pl.ds(0, R)])

    return ek(src3, dst3, G, A, I)


def _mid_body(acc_ref, b1_ref, w2_ref, s2_ref, d2_ref, e8_ref,
              g_ref, a_ref, i_ref):
    At = acc_ref[0, :, 0:80] + acc_ref[1, :, 0:80]
    den = jnp.dot(At[:, 0:8], e8_ref[...],
                  preferred_element_type=jnp.float32) + 1e-16
    h2 = At[:, 16:80] / den + b1_ref[...]
    h2 = jnp.where(h2 > 0, h2, jnp.exp(jnp.minimum(h2, 0.0)) - 1.0)
    z = jnp.dot(h2, w2_ref[...], preferred_element_type=jnp.float32)
    as2 = jnp.dot(z, s2_ref[...], preferred_element_type=jnp.float32)
    ad2 = jnp.dot(z, d2_ref[...], preferred_element_type=jnp.float32)
    sm = as2 + ad2
    w = jnp.exp(jnp.where(sm > 0, sm, 0.2 * sm))
    z15 = jnp.zeros((_NB, 15), jnp.float32)
    z80 = jnp.zeros((_NB, 80), jnp.float32)
    g_ref[...] = jnp.concatenate([as2, z15, z], axis=1)
    a_ref[...] = jnp.concatenate([ad2, z15], axis=1)
    i_ref[...] = jnp.concatenate([0.5 * w, z15, (0.5 * w) * z, z80], axis=1)


def _mid(Acc1, b1, W2, S2T, D2T, E8):
    grid = _N // _NB
    return pl.pallas_call(
        _mid_body,
        grid=(grid,),
        in_specs=[
            pl.BlockSpec((2, _NB, 128), lambda i: (0, i, 0)),
            pl.BlockSpec((1, _H1 * _C1), lambda i: (0, 0)),
            pl.BlockSpec((_H1 * _C1, _NCLS), lambda i: (0, 0)),
            pl.BlockSpec((_NCLS, 1), lambda i: (0, 0)),
            pl.BlockSpec((_NCLS, 1), lambda i: (0, 0)),
            pl.BlockSpec((_H1, _H1 * _C1), lambda i: (0, 0)),
        ],
        out_specs=[
            pl.BlockSpec((_NB, 48), lambda i: (i, 0)),
            pl.BlockSpec((_NB, 16), lambda i: (i, 0)),
            pl.BlockSpec((_NB, 128), lambda i: (i, 0)),
        ],
        out_shape=[
            jax.ShapeDtypeStruct((_N, 48), jnp.float32),
            jax.ShapeDtypeStruct((_N, 16), jnp.float32),
            jax.ShapeDtypeStruct((_N, 128), jnp.float32),
        ],
    )(Acc1, b1, W2, S2T, D2T, E8)


def _fin_body(acc_ref, b2_ref, o_ref):
    At = acc_ref[0, :, 0:48] + acc_ref[1, :, 0:48]
    den = At[:, 0:1] + 1e-16
    logits = At[:, 16:48] / den + b2_ref[...]
    m = jnp.max(logits, axis=1, keepdims=True)
    lse = m + jnp.log(jnp.sum(jnp.exp(logits - m), axis=1, keepdims=True))
    o_ref[...] = logits - lse


def _fin(Acc2, b2):
    grid = _N // _NB
    return pl.pallas_call(
        _fin_body,
        grid=(grid,),
        in_specs=[
            pl.BlockSpec((2, _NB, 128), lambda i: (0, i, 0)),
            pl.BlockSpec((1, _NCLS), lambda i: (0, 0)),
        ],
        out_specs=pl.BlockSpec((_NB, _NCLS), lambda i: (i, 0)),
        out_shape=jax.ShapeDtypeStruct((_N, _NCLS), jnp.float32),
    )(Acc2, b2)


def kernel(x, edge_index, W1, att_src1, att_dst1, b1, W2, att_src2, att_dst2, b2):
    nch = _E // _NW // _K
    src = edge_index[0].reshape(_NW, nch, _K)
    dst = edge_index[1].reshape(_NW, nch, _K)
    eye8 = jnp.eye(_H1, dtype=jnp.float32)
    S1 = (att_src1[:, :, None] * eye8[:, None, :]).reshape(_H1 * _C1, _H1)
    D1 = (att_dst1[:, :, None] * eye8[:, None, :]).reshape(_H1 * _C1, _H1)
    E8 = jnp.repeat(eye8, _C1, axis=1)
    G1, A1, I1 = _prep1(x, W1, S1, D1, E8)
    Acc1 = _edge_pass(src, dst, G1, A1, I1, 80, 4, False)
    G2, A2, I2 = _mid(Acc1, b1.reshape(1, -1), W2,
                      att_src2.reshape(-1, 1), att_dst2.reshape(-1, 1), E8)
    Acc2 = _edge_pass(src, dst, G2, A2, I2, 48, 2, True)
    return _fin(Acc2, b2.reshape(1, -1))


# speedup vs baseline: 1.0432x; 1.0432x over previous
import functools

import jax
import jax.numpy as jnp
from jax import lax
from jax.experimental import pallas as pl
from jax.experimental.pallas import tpu as pltpu
from jax.experimental.pallas import tpu_sc as plsc

_N = 10000
_E = 320000
_F_IN = 128
_H1 = 8
_C1 = 8
_NCLS = 32

_NB = 1000
_NC = 2
_NS = 16
_NW = _NC * _NS
_K = 80
_LANES = 16
_NBUF = 4


def _prep1_body(x_ref, w1_ref, s1_ref, d1_ref, e8_ref, g_ref, a_ref, i_ref):
    h = jnp.dot(x_ref[...], w1_ref[...], preferred_element_type=jnp.float32)
    a_s = jnp.dot(h, s1_ref[...], preferred_element_type=jnp.float32)
    a_d = jnp.dot(h, d1_ref[...], preferred_element_type=jnp.float32)
    z8 = jnp.zeros((_NB, 8), jnp.float32)
    g_ref[...] = jnp.concatenate([a_s, h], axis=1)
    a_ref[...] = jnp.concatenate([a_d, z8], axis=1)
    sm = a_s + a_d
    w = jnp.exp(jnp.where(sm > 0, sm, 0.2 * sm))
    wh = jnp.dot(w, e8_ref[...], preferred_element_type=jnp.float32) * h
    i_ref[...] = jnp.concatenate(
        [0.5 * w, 0.5 * wh, jnp.zeros((_NB, 56), jnp.float32)], axis=1)


def _prep1(x, W1, S1, D1, E8):
    grid = _N // _NB
    return pl.pallas_call(
        _prep1_body,
        grid=(grid,),
        in_specs=[
            pl.BlockSpec((_NB, _F_IN), lambda i: (i, 0)),
            pl.BlockSpec((_F_IN, _H1 * _C1), lambda i: (0, 0)),
            pl.BlockSpec((_H1 * _C1, _H1), lambda i: (0, 0)),
            pl.BlockSpec((_H1 * _C1, _H1), lambda i: (0, 0)),
            pl.BlockSpec((_H1, _H1 * _C1), lambda i: (0, 0)),
        ],
        out_specs=[
            pl.BlockSpec((_NB, 72), lambda i: (i, 0)),
            pl.BlockSpec((_NB, 16), lambda i: (i, 0)),
            pl.BlockSpec((_NB, 128), lambda i: (i, 0)),
        ],
        out_shape=[
            jax.ShapeDtypeStruct((_N, 72), jnp.float32),
            jax.ShapeDtypeStruct((_N, 16), jnp.float32),
            jax.ShapeDtypeStruct((_N, 128), jnp.float32),
        ],
    )(x, W1, S1, D1, E8)


def _edge_pass(src3, dst3, G, A, I, R, nh, bcast):
    n = G.shape[0]
    nch = src3.shape[1]
    r_off = 624
    r_sz = n - r_off * (_NS - 1)
    assert r_off % 8 == 0 and r_sz % 8 == 0 and r_sz >= r_off
    mesh = plsc.VectorSubcoreMesh(core_axis_name="c", subcore_axis_name="s")

    @functools.partial(
        pl.kernel,
        out_type=jax.ShapeDtypeStruct((_NC, n, 128), jnp.float32),
        mesh=mesh,
        compiler_params=pltpu.CompilerParams(
            needs_layout_passes=False, use_tc_tiling_on_sc=False),
        scratch_types=[
            pltpu.VMEM((nch, _K), jnp.int32),
            pltpu.VMEM((nch, _K), jnp.int32),
            pltpu.VMEM((_NBUF, _K, R), jnp.float32),
            pltpu.VMEM((_NBUF, _K, 16), jnp.float32),
            pltpu.VMEM((_NBUF, _K, R), jnp.float32),
            pltpu.VMEM((_K * _LANES,), jnp.float32),
            pltpu.VMEM_SHARED((n, R), jnp.float32),
            pltpu.SemaphoreType.DMA((_NBUF,)),
            pltpu.SemaphoreType.DMA((_NBUF,)),
            pltpu.SemaphoreType.DMA((_NBUF,)),
        ],
    )
    def ek(src_hbm, dst_hbm, g_hbm, a_hbm, i_hbm, out_hbm,
           sidx, didx, grows, arows, mrows, wbuf, acc,
           semg, semr, semd):
        c = lax.axis_index("c")
        s = lax.axis_index("s")
        wid = s * _NC + c
        pltpu.sync_copy(i_hbm.at[pl.ds(s * r_off, r_sz), pl.ds(0, R)],
                        acc.at[pl.ds(s * r_off, r_sz)])
        pltpu.sync_copy(src_hbm.at[wid], sidx)
        pltpu.sync_copy(dst_hbm.at[wid], didx)
        plsc.subcore_barrier()
        iota = lax.iota(jnp.int32, _LANES)

        def issue_gather(i, b):
            pltpu.async_copy(g_hbm.at[sidx.at[i]], grows.at[b], semg.at[b])
            pltpu.async_copy(a_hbm.at[didx.at[i]], arows.at[b], semr.at[b])

        for p in range(_NBUF - 1):
            issue_gather(p, p)

        def chunk(i, carry):
            b = lax.rem(i, _NBUF)

            @pl.when(i + _NBUF - 1 < nch)
            def _():
                issue_gather(i + _NBUF - 1, lax.rem(i + _NBUF - 1, _NBUF))

            pltpu.make_async_copy(
                g_hbm.at[sidx.at[i]], grows.at[b], semg.at[b]).wait()
            pltpu.make_async_copy(
                a_hbm.at[didx.at[i]], arows.at[b], semr.at[b]).wait()

            @pl.when(i >= _NBUF)
            def _():
                pltpu.make_async_copy(
                    mrows.at[b], acc.at[didx.at[i]], semd.at[b]).wait()

            @plsc.parallel_loop(0, _K, unroll=8)
            def edge(e):
                va = grows[b, e, pl.ds(0, 16)]
                vd = arows[b, e, pl.ds(0, 16)]
                sm = va + vd
                w = jnp.exp(jnp.maximum(sm, 0.2 * sm))
                mrows[b, e, pl.ds(0, 16)] = w
                wbuf[pl.ds(e * _LANES, _LANES)] = w
                ebase = jnp.full((_LANES,), e * _LANES, jnp.int32)
                for j in range(nh):
                    if bcast:
                        col = ebase
                    else:
                        col = ebase + (iota >> 3) + (2 * j)
                    wj = plsc.load_gather(wbuf, [col])
                    hj = grows[b, e, pl.ds(8 + 16 * j, 16)]
                    mrows[b, e, pl.ds(8 + 16 * j, 16)] = hj * wj

            pltpu.async_copy(mrows.at[b], acc.at[didx.at[i]],
                             semd.at[b], add=True)
            return carry

        lax.fori_loop(0, nch, chunk, 0)
        for p in range(_NBUF):
            pltpu.make_async_copy(mrows.at[p], acc.at[didx.at[p]],
                                  semd.at[p]).wait()
        plsc.subcore_barrier()
        pltpu.sync_copy(acc.at[pl.ds(s * r_off, r_sz)],
                        out_hbm.at[c, pl.ds(s * r_off, r_sz), pl.ds(0, R)])

    return ek(src3, dst3, G, A, I)


def _mid_body(acc_ref, b1_ref, w2_ref, s2_ref, d2_ref, e8_ref,
              g_ref, a_ref, i_ref):
    At = acc_ref[0, :, 0:72] + acc_ref[1, :, 0:72]
    den = jnp.dot(At[:, 0:8], e8_ref[...],
                  preferred_element_type=jnp.float32) + 1e-16
    h2 = At[:, 8:72] / den + b1_ref[...]
    h2 = jnp.where(h2 > 0, h2, jnp.exp(jnp.minimum(h2, 0.0)) - 1.0)
    z = jnp.dot(h2, w2_ref[...], preferred_element_type=jnp.float32)
    as2 = jnp.dot(z, s2_ref[...], preferred_element_type=jnp.float32)
    ad2 = jnp.dot(z, d2_ref[...], preferred_element_type=jnp.float32)
    sm = as2 + ad2
    w = jnp.exp(jnp.where(sm > 0, sm, 0.2 * sm))
    z7 = jnp.zeros((_NB, 7), jnp.float32)
    z15 = jnp.zeros((_NB, 15), jnp.float32)
    z88 = jnp.zeros((_NB, 88), jnp.float32)
    g_ref[...] = jnp.concatenate([as2, z7, z], axis=1)
    a_ref[...] = jnp.concatenate([ad2, z15], axis=1)
    i_ref[...] = jnp.concatenate([0.5 * w, z7, (0.5 * w) * z, z88], axis=1)


def _mid(Acc1, b1, W2, S2T, D2T, E8):
    grid = _N // _NB
    return pl.pallas_call(
        _mid_body,
        grid=(grid,),
        in_specs=[
            pl.BlockSpec((2, _NB, 128), lambda i: (0, i, 0)),
            pl.BlockSpec((1, _H1 * _C1), lambda i: (0, 0)),
            pl.BlockSpec((_H1 * _C1, _NCLS), lambda i: (0, 0)),
            pl.BlockSpec((_NCLS, 1), lambda i: (0, 0)),
            pl.BlockSpec((_NCLS, 1), lambda i: (0, 0)),
            pl.BlockSpec((_H1, _H1 * _C1), lambda i: (0, 0)),
        ],
        out_specs=[
            pl.BlockSpec((_NB, 40), lambda i: (i, 0)),
            pl.BlockSpec((_NB, 16), lambda i: (i, 0)),
            pl.BlockSpec((_NB, 128), lambda i: (i, 0)),
        ],
        out_shape=[
            jax.ShapeDtypeStruct((_N, 40), jnp.float32),
            jax.ShapeDtypeStruct((_N, 16), jnp.float32),
            jax.ShapeDtypeStruct((_N, 128), jnp.float32),
        ],
    )(Acc1, b1, W2, S2T, D2T, E8)


def _fin_body(acc_ref, b2_ref, o_ref):
    At = acc_ref[0, :, 0:40] + acc_ref[1, :, 0:40]
    den = At[:, 0:1] + 1e-16
    logits = At[:, 8:40] / den + b2_ref[...]
    m = jnp.max(logits, axis=1, keepdims=True)
    lse = m + jnp.log(jnp.sum(jnp.exp(logits - m), axis=1, keepdims=True))
    o_ref[...] = logits - lse


def _fin(Acc2, b2):
    grid = _N // _NB
    return pl.pallas_call(
        _fin_body,
        grid=(grid,),
        in_specs=[
            pl.BlockSpec((2, _NB, 128), lambda i: (0, i, 0)),
            pl.BlockSpec((1, _NCLS), lambda i: (0, 0)),
        ],
        out_specs=pl.BlockSpec((_NB, _NCLS), lambda i: (i, 0)),
        out_shape=jax.ShapeDtypeStruct((_N, _NCLS), jnp.float32),
    )(Acc2, b2)


def kernel(x, edge_index, W1, att_src1, att_dst1, b1, W2, att_src2, att_dst2, b2):
    nch = _E // _NW // _K
    src = edge_index[0].reshape(_NW, nch, _K)
    dst = edge_index[1].reshape(_NW, nch, _K)
    eye8 = jnp.eye(_H1, dtype=jnp.float32)
    S1 = (att_src1[:, :, None] * eye8[:, None, :]).reshape(_H1 * _C1, _H1)
    D1 = (att_dst1[:, :, None] * eye8[:, None, :]).reshape(_H1 * _C1, _H1)
    E8 = jnp.repeat(eye8, _C1, axis=1)
    G1, A1, I1 = _prep1(x, W1, S1, D1, E8)
    Acc1 = _edge_pass(src, dst, G1, A1, I1, 72, 4, False)
    G2, A2, I2 = _mid(Acc1, b1.reshape(1, -1), W2,
                      att_src2.reshape(-1, 1), att_dst2.reshape(-1, 1), E8)
    Acc2 = _edge_pass(src, dst, G2, A2, I2, 40, 2, True)
    return _fin(Acc2, b2.reshape(1, -1))
